# baseline (speedup 1.0000x reference)
import functools

import jax
import jax.numpy as jnp
from jax import lax
from jax.experimental import pallas as pl
from jax.experimental.pallas import tpu as pltpu
from jax.experimental.pallas import tpu_sc as plsc

B = 1024
D = 64
V = 100000
K = 64
T = 2048
NT = 49
VP = NT * T
NS1 = VP // 128
S1 = 80
NG = S1 * 128 // 16
S2 = 80
CAND = S2 * 16
NEG = float(-jnp.finfo(jnp.float32).max)
BIGI = 2**30

BB = 256
NBB = B // BB


def _k1_body(x_ref, cb_ref, wq_ref, bq_ref, wk_ref, bk_ref,
             dots_ref, sub1t_ref, q_s, m1t_s):
    j = pl.program_id(0)

    @pl.when(j == 0)
    def _():
        q_s[...] = lax.dot_general(
            x_ref[...], wq_ref[...],
            dimension_numbers=(((1,), (1,)), ((), ())),
            preferred_element_type=jnp.float32) + bq_ref[...]

    k_t = lax.dot_general(
        cb_ref[...], wk_ref[...],
        dimension_numbers=(((1,), (1,)), ((), ())),
        preferred_element_type=jnp.float32) + bk_ref[...]
    dots = lax.dot_general(
        q_s[...], k_t,
        dimension_numbers=(((1,), (1,)), ((), ())),
        preferred_element_type=jnp.float32)

    @pl.when(j < NT - 1)
    def _():
        dots_ref[...] = dots
        m1t_s[pl.ds(j * (T // 128), T // 128), :] = jnp.transpose(
            jnp.max(dots.reshape(B, T // 128, 128), axis=2))

    @pl.when(j == NT - 1)
    def _():
        col = j * T + lax.broadcasted_iota(jnp.int32, (B, T), 1)
        dm = jnp.where(col < V, dots, NEG)
        dots_ref[...] = dm
        m1t_s[pl.ds(j * (T // 128), T // 128), :] = jnp.transpose(
            jnp.max(dm.reshape(B, T // 128, 128), axis=2))

        sub1t_ref[...] = jnp.zeros((S1, B), jnp.int32)
        li = lax.broadcasted_iota(jnp.int32, (NS1, B), 0)
        lo = lax.broadcasted_iota(jnp.int32, (S1, B), 0)

        def body(t, _):
            cur = m1t_s[...]
            m = jnp.max(cur, axis=0, keepdims=True)
            im = jnp.min(jnp.where(cur >= m, li, BIGI), axis=0, keepdims=True)
            sub1t_ref[...] += jnp.where(lo == t, im, 0)
            m1t_s[...] = jnp.where(li == im, NEG, cur)
            return 0

        lax.fori_loop(0, S1, body, 0)


def _k1(x, cb, wq, bq, wk, bk):
    return pl.pallas_call(
        _k1_body,
        grid=(NT,),
        in_specs=[
            pl.BlockSpec((B, D), lambda j: (0, 0)),
            pl.BlockSpec((T, D), lambda j: (j, 0)),
            pl.BlockSpec((D, D), lambda j: (0, 0)),
            pl.BlockSpec((1, D), lambda j: (0, 0)),
            pl.BlockSpec((D, D), lambda j: (0, 0)),
            pl.BlockSpec((1, D), lambda j: (0, 0)),
        ],
        out_specs=[
            pl.BlockSpec((B, T), lambda j: (0, j)),
            pl.BlockSpec((S1, B), lambda j: (0, 0)),
        ],
        out_shape=[
            jax.ShapeDtypeStruct((B, VP), jnp.float32),
            jax.ShapeDtypeStruct((S1, B), jnp.int32),
        ],
        scratch_shapes=[
            pltpu.VMEM((B, D), jnp.float32),
            pltpu.VMEM((NS1, B), jnp.float32),
        ],
    )(x, cb, wq, bq, wk, bk)


def _sc_gather(table, idx, row_w, n_idx):
    info = plsc.get_sparse_core_info()
    nw = info.num_cores * info.num_subcores
    per_w = n_idx // nw
    chunks = per_w // 128
    mesh = plsc.VectorSubcoreMesh(core_axis_name="c", subcore_axis_name="s")

    @functools.partial(
        pl.kernel, mesh=mesh,
        out_type=jax.ShapeDtypeStruct((n_idx, row_w), jnp.float32),
        scratch_types=[
            pltpu.VMEM((chunks, 128), jnp.int32),
            pltpu.VMEM((128, row_w), jnp.float32),
            pltpu.VMEM((128, row_w), jnp.float32),
            pltpu.SemaphoreType.DMA,
            pltpu.SemaphoreType.DMA,
        ],
    )
    def kern(table_hbm, idx_hbm, out_hbm, idx_v, rows0, rows1, sem0, sem1):
        wid = lax.axis_index("s") * info.num_cores + lax.axis_index("c")
        pltpu.sync_copy(idx_hbm.at[wid], idx_v)
        bufs = (rows0, rows1)
        sems = (sem0, sem1)

        def start(ci, b):
            @pl.when(ci < chunks)
            def _():
                pltpu.make_async_copy(
                    table_hbm.at[idx_v.at[ci]], bufs[b], sems[b]).start()

        def drain_store(ci, b):
            pltpu.make_async_copy(
                table_hbm.at[idx_v.at[ci]], bufs[b], sems[b]).wait()
            pltpu.sync_copy(
                bufs[b], out_hbm.at[pl.ds((wid * chunks + ci) * 128, 128)])

        start(0, 0)

        def pair(cb, _):
            c0 = cb * 2
            start(c0 + 1, 1)
            drain_store(c0, 0)
            start(c0 + 2, 0)
            drain_store(c0 + 1, 1)
            return 0

        lax.fori_loop(0, chunks // 2, pair, 0)

    return kern(table, idx.reshape(nw, chunks, 128))


def _k3_body(cand_ref, sub1t_ref, idx2t_ref, gbt_ref, cur_s):
    i = pl.program_id(0)
    NCH = 4
    rows = NG // NCH
    for c in range(NCH):
        blk = jnp.transpose(
            cand_ref[:, c * rows * 16:(c + 1) * rows * 16])
        cur_s[c * rows:(c + 1) * rows, :] = jnp.max(
            blk.reshape(rows, 16, BB), axis=1)
    idx2t_ref[...] = jnp.zeros((S2, BB), jnp.int32)
    li = lax.broadcasted_iota(jnp.int32, (NG, BB), 0)
    lo = lax.broadcasted_iota(jnp.int32, (S2, BB), 0)

    def body(t, _):
        cur = cur_s[...]
        m = jnp.max(cur, axis=0, keepdims=True)
        im = jnp.min(jnp.where(cur >= m, li, BIGI), axis=0, keepdims=True)
        idx2t_ref[...] += jnp.where(lo == t, im, 0)
        cur_s[...] = jnp.where(li == im, NEG, cur)
        return 0

    lax.fori_loop(0, S2, body, 0)

    sub2 = idx2t_ref[...]
    g_hi = sub2 // 8
    g_lo = sub2 % 8
    sub1 = sub1t_ref[...]
    UCH = 16
    parts = []
    for c in range(S2 // UCH):
        gh = g_hi[c * UCH:(c + 1) * UCH, :]
        oh = gh[:, None, :] == lax.broadcasted_iota(
            jnp.int32, (1, S1, 1), 1)
        parts.append(jnp.sum(jnp.where(oh, sub1[None, :, :], 0), axis=1))
    s1sel = jnp.concatenate(parts, axis=0)
    gbt_ref[...] = s1sel * 128 + g_lo * 16
    br = i * BB + lax.broadcasted_iota(jnp.int32, (S2, BB), 1)
    idx2t_ref[...] = br * S1 + g_hi


def _k3(cand_nat, sub1t):
    return pl.pallas_call(
        _k3_body,
        grid=(NBB,),
        in_specs=[
            pl.BlockSpec((BB, S1 * 128), lambda i: (i, 0)),
            pl.BlockSpec((S1, BB), lambda i: (0, i)),
        ],
        out_specs=[
            pl.BlockSpec((S2, BB), lambda i: (0, i)),
            pl.BlockSpec((S2, BB), lambda i: (0, i)),
        ],
        out_shape=[
            jax.ShapeDtypeStruct((S2, B), jnp.int32),
            jax.ShapeDtypeStruct((S2, B), jnp.int32),
        ],
        scratch_shapes=[pltpu.VMEM((NG, BB), jnp.float32)],
    )(cand_nat, sub1t)


def _k4_body(cand2t_ref, gbt_ref, tv_ref, ti_ref, attn_ref, vk_ref,
             cur_s, gidx_s):
    gb = gbt_ref[...]
    g_lo = (gb // 16) % 8
    UCH = 10
    for c in range(S2 // UCH):
        cw = jnp.transpose(
            cand2t_ref[:, c * UCH * 128:(c + 1) * UCH * 128]).reshape(
                UCH, 8, 16, BB)
        sel = (g_lo[c * UCH:(c + 1) * UCH, None, :] ==
               lax.broadcasted_iota(jnp.int32, (1, 8, 1), 1))
        cc = jnp.max(jnp.where(sel[:, :, None, :], cw, NEG), axis=1)
        cur_s[c * UCH * 16:(c + 1) * UCH * 16, :] = cc.reshape(UCH * 16, BB)
    t16 = lax.broadcasted_iota(jnp.int32, (S2, 16, BB), 1)
    gidx_s[...] = (gb[:, None, :] + t16).reshape(CAND, BB)
    tv_ref[...] = jnp.zeros((K, BB), jnp.float32)
    ti_ref[...] = jnp.zeros((K, BB), jnp.int32)
    lo = lax.broadcasted_iota(jnp.int32, (K, BB), 0)

    def body(t, _):
        cur = cur_s[...]
        gidx = gidx_s[...]
        m = jnp.max(cur, axis=0, keepdims=True)
        gi = jnp.min(jnp.where(cur >= m, gidx, BIGI), axis=0, keepdims=True)
        sel = lo == t
        tv_ref[...] += jnp.where(sel, m, 0.0)
        ti_ref[...] += jnp.where(sel, gi, 0)
        cur_s[...] = jnp.where(gidx == gi, NEG, cur)
        return 0

    lax.fori_loop(0, K, body, 0)

    tv = tv_ref[...]
    e = jnp.exp(tv - tv[0:1, :])
    attn_ref[...] = e / jnp.sum(e, axis=0, keepdims=True)
    vk_ref[...] = tv[K - 1:K, :]


def _k4(cand2_t, gbt):
    return pl.pallas_call(
        _k4_body,
        grid=(NBB,),
        in_specs=[
            pl.BlockSpec((BB, S2 * 128), lambda i: (i, 0)),
            pl.BlockSpec((S2, BB), lambda i: (0, i)),
        ],
        out_specs=[
            pl.BlockSpec((K, BB), lambda i: (0, i)),
            pl.BlockSpec((K, BB), lambda i: (0, i)),
            pl.BlockSpec((K, BB), lambda i: (0, i)),
            pl.BlockSpec((1, BB), lambda i: (0, i)),
        ],
        out_shape=[
            jax.ShapeDtypeStruct((K, B), jnp.float32),
            jax.ShapeDtypeStruct((K, B), jnp.int32),
            jax.ShapeDtypeStruct((K, B), jnp.float32),
            jax.ShapeDtypeStruct((1, B), jnp.float32),
        ],
        scratch_shapes=[
            pltpu.VMEM((CAND, BB), jnp.float32),
            pltpu.VMEM((CAND, BB), jnp.int32),
        ],
    )(cand2_t, gbt)


RB = 8192


def _k5_body(rows_ref, wv_ref, bv_ref, attn_ref, par_ref, out_ref):
    nb = RB // K
    r3 = rows_ref[...].reshape(nb, K, 2 * D)
    sel = par_ref[...][:, :, None] == 1
    cbr = jnp.where(sel, r3[:, :, D:], r3[:, :, :D]).reshape(RB, D)
    v = lax.dot_general(
        cbr, wv_ref[...],
        dimension_numbers=(((1,), (1,)), ((), ())),
        preferred_element_type=jnp.float32) + bv_ref[...]
    attn_t = jnp.tile(attn_ref[...], (1, nb))
    seg = lax.broadcasted_iota(jnp.int32, (nb, RB), 1) // K
    bl = lax.broadcasted_iota(jnp.int32, (nb, RB), 0)
    w = jnp.where(seg == bl, attn_t, 0.0)
    out_ref[...] = lax.dot_general(
        w, v, dimension_numbers=(((1,), (0,)), ((), ())),
        preferred_element_type=jnp.float32)


def _k5(rows, wv, bv, attn, par):
    nb = (B * K) // RB
    return pl.pallas_call(
        _k5_body,
        grid=(nb,),
        in_specs=[
            pl.BlockSpec((RB, 2 * D), lambda i: (i, 0)),
            pl.BlockSpec((D, D), lambda i: (0, 0)),
            pl.BlockSpec((1, D), lambda i: (0, 0)),
            pl.BlockSpec((RB // K, K), lambda i: (i, 0)),
            pl.BlockSpec((RB // K, K), lambda i: (i, 0)),
        ],
        out_specs=pl.BlockSpec((RB // K, D), lambda i: (i, 0)),
        out_shape=jax.ShapeDtypeStruct((B, D), jnp.float32),
    )(rows, wv, bv, attn, par)


def _k6_body(x_ref, cb_ref, wq_ref, bq_ref, wk_ref, bk_ref, vk_ref,
             out_ref, q_s):
    j = pl.program_id(0)

    @pl.when(j == 0)
    def _():
        q_s[...] = lax.dot_general(
            x_ref[...], wq_ref[...],
            dimension_numbers=(((1,), (1,)), ((), ())),
            preferred_element_type=jnp.float32) + bq_ref[...]

    k_t = lax.dot_general(
        cb_ref[...], wk_ref[...],
        dimension_numbers=(((1,), (1,)), ((), ())),
        preferred_element_type=jnp.float32) + bk_ref[...]
    dots = lax.dot_general(
        q_s[...], k_t,
        dimension_numbers=(((1,), (1,)), ((), ())),
        preferred_element_type=jnp.float32)
    out_ref[...] = jnp.where(dots < vk_ref[...], NEG, dots)


def _k6(x, cb, wq, bq, wk, bk, vk):
    return pl.pallas_call(
        _k6_body,
        grid=(NT,),
        in_specs=[
            pl.BlockSpec((B, D), lambda j: (0, 0)),
            pl.BlockSpec((T, D), lambda j: (j, 0)),
            pl.BlockSpec((D, D), lambda j: (0, 0)),
            pl.BlockSpec((1, D), lambda j: (0, 0)),
            pl.BlockSpec((D, D), lambda j: (0, 0)),
            pl.BlockSpec((1, D), lambda j: (0, 0)),
            pl.BlockSpec((B, 1), lambda j: (0, 0)),
        ],
        out_specs=pl.BlockSpec((B, T), lambda j: (0, j)),
        out_shape=jax.ShapeDtypeStruct((B, V), jnp.float32),
        scratch_shapes=[pltpu.VMEM((B, D), jnp.float32)],
    )(x, cb, wq, bq, wk, bk, vk)


def kernel(x, codebook, Wq, bq, Wk, bk, Wv, bv):
    dots_scratch, sub1t = _k1(x, codebook, Wq, bq.reshape(1, D),
                              Wk, bk.reshape(1, D))
    sub1 = sub1t.T
    idx1 = sub1 + jnp.arange(B, dtype=jnp.int32)[:, None] * NS1
    cand = _sc_gather(dots_scratch.reshape(B * NS1, 128),
                      idx1.reshape(B * S1), 128, B * S1)
    idx2t, gbt = _k3(cand.reshape(B, S1 * 128), sub1t)
    cand2 = _sc_gather(cand, idx2t.T.reshape(B * S2), 128, B * S2)
    tvt, tit, attnt, vkt = _k4(cand2.reshape(B, S2 * 128), gbt)
    top_value, top_ind, attn, vk = tvt.T, tit.T, attnt.T, vkt.T
    cb2 = codebook.reshape(V // 2, 2 * D)
    rows = _sc_gather(cb2, (top_ind // 2).reshape(B * K), 2 * D, B * K)
    out = _k5(rows, Wv, bv.reshape(1, D), attn, top_ind % 2)
    dots_masked = _k6(x, codebook, Wq, bq.reshape(1, D), Wk, bk.reshape(1, D),
                      vk)
    return (out, top_value, top_ind, dots_masked)


# speedup vs baseline: 1.0331x; 1.0331x over previous
import functools

import jax
import jax.numpy as jnp
from jax import lax
from jax.experimental import pallas as pl
from jax.experimental.pallas import tpu as pltpu
from jax.experimental.pallas import tpu_sc as plsc

B = 1024
D = 64
V = 100000
K = 64
T = 2048
NT = 49
VP = NT * T
NS1 = VP // 128
S1 = 72
NG = S1 * 128 // 16
S2 = 72
CAND = S2 * 16
NEG = float(-jnp.finfo(jnp.float32).max)
BIGI = 2**30

BB = 256
NBB = B // BB


def _k1_body(x_ref, cb_ref, wq_ref, bq_ref, wk_ref, bk_ref,
             dots_ref, sub1t_ref, q_s, m1t_s):
    j = pl.program_id(0)

    @pl.when(j == 0)
    def _():
        q_s[...] = lax.dot_general(
            x_ref[...], wq_ref[...],
            dimension_numbers=(((1,), (1,)), ((), ())),
            preferred_element_type=jnp.float32) + bq_ref[...]

    k_t = lax.dot_general(
        cb_ref[...], wk_ref[...],
        dimension_numbers=(((1,), (1,)), ((), ())),
        preferred_element_type=jnp.float32) + bk_ref[...]
    dots = lax.dot_general(
        q_s[...], k_t,
        dimension_numbers=(((1,), (1,)), ((), ())),
        preferred_element_type=jnp.float32)

    @pl.when(j < NT - 1)
    def _():
        dots_ref[...] = dots
        m1t_s[pl.ds(j * (T // 128), T // 128), :] = jnp.transpose(
            jnp.max(dots.reshape(B, T // 128, 128), axis=2))

    @pl.when(j == NT - 1)
    def _():
        col = j * T + lax.broadcasted_iota(jnp.int32, (B, T), 1)
        dm = jnp.where(col < V, dots, NEG)
        dots_ref[...] = dm
        m1t_s[pl.ds(j * (T // 128), T // 128), :] = jnp.transpose(
            jnp.max(dm.reshape(B, T // 128, 128), axis=2))

        sub1t_ref[...] = jnp.zeros((S1, B), jnp.int32)
        li = lax.broadcasted_iota(jnp.int32, (NS1, B), 0)
        lo = lax.broadcasted_iota(jnp.int32, (S1, B), 0)

        def body(t, _):
            cur = m1t_s[...]
            m = jnp.max(cur, axis=0, keepdims=True)
            im = jnp.min(jnp.where(cur >= m, li, BIGI), axis=0, keepdims=True)
            sub1t_ref[...] += jnp.where(lo == t, im, 0)
            m1t_s[...] = jnp.where(li == im, NEG, cur)
            return 0

        lax.fori_loop(0, S1, body, 0)


def _k1(x, cb, wq, bq, wk, bk):
    return pl.pallas_call(
        _k1_body,
        grid=(NT,),
        in_specs=[
            pl.BlockSpec((B, D), lambda j: (0, 0)),
            pl.BlockSpec((T, D), lambda j: (j, 0)),
            pl.BlockSpec((D, D), lambda j: (0, 0)),
            pl.BlockSpec((1, D), lambda j: (0, 0)),
            pl.BlockSpec((D, D), lambda j: (0, 0)),
            pl.BlockSpec((1, D), lambda j: (0, 0)),
        ],
        out_specs=[
            pl.BlockSpec((B, T), lambda j: (0, j)),
            pl.BlockSpec((S1, B), lambda j: (0, 0)),
        ],
        out_shape=[
            jax.ShapeDtypeStruct((B, VP), jnp.float32),
            jax.ShapeDtypeStruct((S1, B), jnp.int32),
        ],
        scratch_shapes=[
            pltpu.VMEM((B, D), jnp.float32),
            pltpu.VMEM((NS1, B), jnp.float32),
        ],
    )(x, cb, wq, bq, wk, bk)


def _sc_gather(table, idx, row_w, n_idx):
    info = plsc.get_sparse_core_info()
    nw = info.num_cores * info.num_subcores
    per_w = n_idx // nw
    chunks = per_w // 128
    mesh = plsc.VectorSubcoreMesh(core_axis_name="c", subcore_axis_name="s")

    @functools.partial(
        pl.kernel, mesh=mesh,
        out_type=jax.ShapeDtypeStruct((n_idx, row_w), jnp.float32),
        scratch_types=[
            pltpu.VMEM((chunks, 128), jnp.int32),
            pltpu.VMEM((128, row_w), jnp.float32),
            pltpu.VMEM((128, row_w), jnp.float32),
            pltpu.SemaphoreType.DMA,
            pltpu.SemaphoreType.DMA,
        ],
    )
    def kern(table_hbm, idx_hbm, out_hbm, idx_v, rows0, rows1, sem0, sem1):
        wid = lax.axis_index("s") * info.num_cores + lax.axis_index("c")
        pltpu.sync_copy(idx_hbm.at[wid], idx_v)
        bufs = (rows0, rows1)
        sems = (sem0, sem1)

        def start(ci, b):
            @pl.when(ci < chunks)
            def _():
                pltpu.make_async_copy(
                    table_hbm.at[idx_v.at[ci]], bufs[b], sems[b]).start()

        def drain_store(ci, b):
            pltpu.make_async_copy(
                table_hbm.at[idx_v.at[ci]], bufs[b], sems[b]).wait()
            pltpu.sync_copy(
                bufs[b], out_hbm.at[pl.ds((wid * chunks + ci) * 128, 128)])

        start(0, 0)

        def pair(cb, _):
            c0 = cb * 2
            start(c0 + 1, 1)
            drain_store(c0, 0)
            start(c0 + 2, 0)
            drain_store(c0 + 1, 1)
            return 0

        lax.fori_loop(0, chunks // 2, pair, 0)

    return kern(table, idx.reshape(nw, chunks, 128))


def _k3_body(cand_ref, sub1t_ref, idx2t_ref, gbt_ref, cur_s):
    i = pl.program_id(0)
    NCH = 4
    rows = NG // NCH
    for c in range(NCH):
        blk = jnp.transpose(
            cand_ref[:, c * rows * 16:(c + 1) * rows * 16])
        cur_s[c * rows:(c + 1) * rows, :] = jnp.max(
            blk.reshape(rows, 16, BB), axis=1)
    idx2t_ref[...] = jnp.zeros((S2, BB), jnp.int32)
    li = lax.broadcasted_iota(jnp.int32, (NG, BB), 0)
    lo = lax.broadcasted_iota(jnp.int32, (S2, BB), 0)

    def body(t, _):
        cur = cur_s[...]
        m = jnp.max(cur, axis=0, keepdims=True)
        im = jnp.min(jnp.where(cur >= m, li, BIGI), axis=0, keepdims=True)
        idx2t_ref[...] += jnp.where(lo == t, im, 0)
        cur_s[...] = jnp.where(li == im, NEG, cur)
        return 0

    lax.fori_loop(0, S2, body, 0)

    sub2 = idx2t_ref[...]
    g_hi = sub2 // 8
    g_lo = sub2 % 8
    sub1 = sub1t_ref[...]
    UCH = 12
    parts = []
    for c in range(S2 // UCH):
        gh = g_hi[c * UCH:(c + 1) * UCH, :]
        oh = gh[:, None, :] == lax.broadcasted_iota(
            jnp.int32, (1, S1, 1), 1)
        parts.append(jnp.sum(jnp.where(oh, sub1[None, :, :], 0), axis=1))
    s1sel = jnp.concatenate(parts, axis=0)
    gbt_ref[...] = s1sel * 128 + g_lo * 16
    br = i * BB + lax.broadcasted_iota(jnp.int32, (S2, BB), 1)
    idx2t_ref[...] = br * S1 + g_hi


def _k3(cand_nat, sub1t):
    return pl.pallas_call(
        _k3_body,
        grid=(NBB,),
        in_specs=[
            pl.BlockSpec((BB, S1 * 128), lambda i: (i, 0)),
            pl.BlockSpec((S1, BB), lambda i: (0, i)),
        ],
        out_specs=[
            pl.BlockSpec((S2, BB), lambda i: (0, i)),
            pl.BlockSpec((S2, BB), lambda i: (0, i)),
        ],
        out_shape=[
            jax.ShapeDtypeStruct((S2, B), jnp.int32),
            jax.ShapeDtypeStruct((S2, B), jnp.int32),
        ],
        scratch_shapes=[pltpu.VMEM((NG, BB), jnp.float32)],
    )(cand_nat, sub1t)


def _k4_body(cand2t_ref, gbt_ref, tv_ref, ti_ref, attn_ref, vk_ref,
             cur_s, gidx_s):
    gb = gbt_ref[...]
    g_lo = (gb // 16) % 8
    UCH = 12
    for c in range(S2 // UCH):
        cw = jnp.transpose(
            cand2t_ref[:, c * UCH * 128:(c + 1) * UCH * 128]).reshape(
                UCH, 8, 16, BB)
        sel = (g_lo[c * UCH:(c + 1) * UCH, None, :] ==
               lax.broadcasted_iota(jnp.int32, (1, 8, 1), 1))
        cc = jnp.max(jnp.where(sel[:, :, None, :], cw, NEG), axis=1)
        cur_s[c * UCH * 16:(c + 1) * UCH * 16, :] = cc.reshape(UCH * 16, BB)
    t16 = lax.broadcasted_iota(jnp.int32, (S2, 16, BB), 1)
    gidx_s[...] = (gb[:, None, :] + t16).reshape(CAND, BB)
    tv_ref[...] = jnp.zeros((K, BB), jnp.float32)
    ti_ref[...] = jnp.zeros((K, BB), jnp.int32)
    lo = lax.broadcasted_iota(jnp.int32, (K, BB), 0)

    def body(t, _):
        cur = cur_s[...]
        gidx = gidx_s[...]
        m = jnp.max(cur, axis=0, keepdims=True)
        gi = jnp.min(jnp.where(cur >= m, gidx, BIGI), axis=0, keepdims=True)
        sel = lo == t
        tv_ref[...] += jnp.where(sel, m, 0.0)
        ti_ref[...] += jnp.where(sel, gi, 0)
        cur_s[...] = jnp.where(gidx == gi, NEG, cur)
        return 0

    lax.fori_loop(0, K, body, 0)

    tv = tv_ref[...]
    e = jnp.exp(tv - tv[0:1, :])
    attn_ref[...] = e / jnp.sum(e, axis=0, keepdims=True)
    vk_ref[...] = tv[K - 1:K, :]


def _k4(cand2_t, gbt):
    return pl.pallas_call(
        _k4_body,
        grid=(NBB,),
        in_specs=[
            pl.BlockSpec((BB, S2 * 128), lambda i: (i, 0)),
            pl.BlockSpec((S2, BB), lambda i: (0, i)),
        ],
        out_specs=[
            pl.BlockSpec((K, BB), lambda i: (0, i)),
            pl.BlockSpec((K, BB), lambda i: (0, i)),
            pl.BlockSpec((K, BB), lambda i: (0, i)),
            pl.BlockSpec((1, BB), lambda i: (0, i)),
        ],
        out_shape=[
            jax.ShapeDtypeStruct((K, B), jnp.float32),
            jax.ShapeDtypeStruct((K, B), jnp.int32),
            jax.ShapeDtypeStruct((K, B), jnp.float32),
            jax.ShapeDtypeStruct((1, B), jnp.float32),
        ],
        scratch_shapes=[
            pltpu.VMEM((CAND, BB), jnp.float32),
            pltpu.VMEM((CAND, BB), jnp.int32),
        ],
    )(cand2_t, gbt)


RB = 8192


def _k5_body(rows_ref, wv_ref, bv_ref, attn_ref, par_ref, out_ref):
    nb = RB // K
    r3 = rows_ref[...].reshape(nb, K, 2 * D)
    sel = par_ref[...][:, :, None] == 1
    cbr = jnp.where(sel, r3[:, :, D:], r3[:, :, :D]).reshape(RB, D)
    v = lax.dot_general(
        cbr, wv_ref[...],
        dimension_numbers=(((1,), (1,)), ((), ())),
        preferred_element_type=jnp.float32) + bv_ref[...]
    attn_t = jnp.tile(attn_ref[...], (1, nb))
    seg = lax.broadcasted_iota(jnp.int32, (nb, RB), 1) // K
    bl = lax.broadcasted_iota(jnp.int32, (nb, RB), 0)
    w = jnp.where(seg == bl, attn_t, 0.0)
    out_ref[...] = lax.dot_general(
        w, v, dimension_numbers=(((1,), (0,)), ((), ())),
        preferred_element_type=jnp.float32)


def _k5(rows, wv, bv, attn, par):
    nb = (B * K) // RB
    return pl.pallas_call(
        _k5_body,
        grid=(nb,),
        in_specs=[
            pl.BlockSpec((RB, 2 * D), lambda i: (i, 0)),
            pl.BlockSpec((D, D), lambda i: (0, 0)),
            pl.BlockSpec((1, D), lambda i: (0, 0)),
            pl.BlockSpec((RB // K, K), lambda i: (i, 0)),
            pl.BlockSpec((RB // K, K), lambda i: (i, 0)),
        ],
        out_specs=pl.BlockSpec((RB // K, D), lambda i: (i, 0)),
        out_shape=jax.ShapeDtypeStruct((B, D), jnp.float32),
    )(rows, wv, bv, attn, par)


def _k6_body(x_ref, cb_ref, wq_ref, bq_ref, wk_ref, bk_ref, vk_ref,
             out_ref, q_s):
    j = pl.program_id(0)

    @pl.when(j == 0)
    def _():
        q_s[...] = lax.dot_general(
            x_ref[...], wq_ref[...],
            dimension_numbers=(((1,), (1,)), ((), ())),
            preferred_element_type=jnp.float32) + bq_ref[...]

    k_t = lax.dot_general(
        cb_ref[...], wk_ref[...],
        dimension_numbers=(((1,), (1,)), ((), ())),
        preferred_element_type=jnp.float32) + bk_ref[...]
    dots = lax.dot_general(
        q_s[...], k_t,
        dimension_numbers=(((1,), (1,)), ((), ())),
        preferred_element_type=jnp.float32)
    out_ref[...] = jnp.where(dots < vk_ref[...], NEG, dots)


def _k6(x, cb, wq, bq, wk, bk, vk):
    return pl.pallas_call(
        _k6_body,
        grid=(NT,),
        in_specs=[
            pl.BlockSpec((B, D), lambda j: (0, 0)),
            pl.BlockSpec((T, D), lambda j: (j, 0)),
            pl.BlockSpec((D, D), lambda j: (0, 0)),
            pl.BlockSpec((1, D), lambda j: (0, 0)),
            pl.BlockSpec((D, D), lambda j: (0, 0)),
            pl.BlockSpec((1, D), lambda j: (0, 0)),
            pl.BlockSpec((B, 1), lambda j: (0, 0)),
        ],
        out_specs=pl.BlockSpec((B, T), lambda j: (0, j)),
        out_shape=jax.ShapeDtypeStruct((B, V), jnp.float32),
        scratch_shapes=[pltpu.VMEM((B, D), jnp.float32)],
    )(x, cb, wq, bq, wk, bk, vk)


def kernel(x, codebook, Wq, bq, Wk, bk, Wv, bv):
    dots_scratch, sub1t = _k1(x, codebook, Wq, bq.reshape(1, D),
                              Wk, bk.reshape(1, D))
    sub1 = sub1t.T
    idx1 = sub1 + jnp.arange(B, dtype=jnp.int32)[:, None] * NS1
    cand = _sc_gather(dots_scratch.reshape(B * NS1, 128),
                      idx1.reshape(B * S1), 128, B * S1)
    idx2t, gbt = _k3(cand.reshape(B, S1 * 128), sub1t)
    cand2 = _sc_gather(cand, idx2t.T.reshape(B * S2), 128, B * S2)
    tvt, tit, attnt, vkt = _k4(cand2.reshape(B, S2 * 128), gbt)
    top_value, top_ind, attn, vk = tvt.T, tit.T, attnt.T, vkt.T
    cb2 = codebook.reshape(V // 2, 2 * D)
    rows = _sc_gather(cb2, (top_ind // 2).reshape(B * K), 2 * D, B * K)
    out = _k5(rows, Wv, bv.reshape(1, D), attn, top_ind % 2)
    dots_masked = _k6(x, codebook, Wq, bq.reshape(1, D), Wk, bk.reshape(1, D),
                      vk)
    return (out, top_value, top_ind, dots_masked)


# speedup vs baseline: 1.0417x; 1.0083x over previous
import functools

import jax
import jax.numpy as jnp
from jax import lax
from jax.experimental import pallas as pl
from jax.experimental.pallas import tpu as pltpu
from jax.experimental.pallas import tpu_sc as plsc

B = 1024
D = 64
V = 100000
K = 64
T = 3072
NT = 33
VP = NT * T
NS1 = VP // 128
S1 = 72
NG = S1 * 128 // 16
S2 = 72
CAND = S2 * 16
NEG = float(-jnp.finfo(jnp.float32).max)
BIGI = 2**30

BB = 256
NBB = B // BB


def _k1_body(x_ref, cb_ref, wq_ref, bq_ref, wk_ref, bk_ref,
             dots_ref, sub1t_ref, q_s, m1t_s):
    j = pl.program_id(0)

    @pl.when(j == 0)
    def _():
        q_s[...] = lax.dot_general(
            x_ref[...], wq_ref[...],
            dimension_numbers=(((1,), (1,)), ((), ())),
            preferred_element_type=jnp.float32) + bq_ref[...]

    k_t = lax.dot_general(
        cb_ref[...], wk_ref[...],
        dimension_numbers=(((1,), (1,)), ((), ())),
        preferred_element_type=jnp.float32) + bk_ref[...]
    dots = lax.dot_general(
        q_s[...], k_t,
        dimension_numbers=(((1,), (1,)), ((), ())),
        preferred_element_type=jnp.float32)

    @pl.when(j < NT - 1)
    def _():
        dots_ref[...] = dots
        m1t_s[pl.ds(j * (T // 128), T // 128), :] = jnp.transpose(
            jnp.max(dots.reshape(B, T // 128, 128), axis=2))

    @pl.when(j == NT - 1)
    def _():
        col = j * T + lax.broadcasted_iota(jnp.int32, (B, T), 1)
        dm = jnp.where(col < V, dots, NEG)
        dots_ref[...] = dm
        m1t_s[pl.ds(j * (T // 128), T // 128), :] = jnp.transpose(
            jnp.max(dm.reshape(B, T // 128, 128), axis=2))

        sub1t_ref[...] = jnp.zeros((S1, B), jnp.int32)
        li = lax.broadcasted_iota(jnp.int32, (NS1, B), 0)
        lo = lax.broadcasted_iota(jnp.int32, (S1, B), 0)

        def body(t, _):
            cur = m1t_s[...]
            m = jnp.max(cur, axis=0, keepdims=True)
            im = jnp.min(jnp.where(cur >= m, li, BIGI), axis=0, keepdims=True)
            sub1t_ref[...] += jnp.where(lo == t, im, 0)
            m1t_s[...] = jnp.where(li == im, NEG, cur)
            return 0

        lax.fori_loop(0, S1, body, 0)


def _k1(x, cb, wq, bq, wk, bk):
    return pl.pallas_call(
        _k1_body,
        grid=(NT,),
        in_specs=[
            pl.BlockSpec((B, D), lambda j: (0, 0)),
            pl.BlockSpec((T, D), lambda j: (j, 0)),
            pl.BlockSpec((D, D), lambda j: (0, 0)),
            pl.BlockSpec((1, D), lambda j: (0, 0)),
            pl.BlockSpec((D, D), lambda j: (0, 0)),
            pl.BlockSpec((1, D), lambda j: (0, 0)),
        ],
        out_specs=[
            pl.BlockSpec((B, T), lambda j: (0, j)),
            pl.BlockSpec((S1, B), lambda j: (0, 0)),
        ],
        out_shape=[
            jax.ShapeDtypeStruct((B, VP), jnp.float32),
            jax.ShapeDtypeStruct((S1, B), jnp.int32),
        ],
        scratch_shapes=[
            pltpu.VMEM((B, D), jnp.float32),
            pltpu.VMEM((NS1, B), jnp.float32),
        ],
    )(x, cb, wq, bq, wk, bk)


def _sc_gather(table, idx, row_w, n_idx):
    info = plsc.get_sparse_core_info()
    nw = info.num_cores * info.num_subcores
    per_w = n_idx // nw
    chunks = per_w // 128
    mesh = plsc.VectorSubcoreMesh(core_axis_name="c", subcore_axis_name="s")

    @functools.partial(
        pl.kernel, mesh=mesh,
        out_type=jax.ShapeDtypeStruct((n_idx, row_w), jnp.float32),
        scratch_types=[
            pltpu.VMEM((chunks, 128), jnp.int32),
            pltpu.VMEM((128, row_w), jnp.float32),
            pltpu.VMEM((128, row_w), jnp.float32),
            pltpu.VMEM((128, row_w), jnp.float32),
            pltpu.VMEM((128, row_w), jnp.float32),
            pltpu.SemaphoreType.DMA,
            pltpu.SemaphoreType.DMA,
            pltpu.SemaphoreType.DMA,
            pltpu.SemaphoreType.DMA,
        ],
    )
    def kern(table_hbm, idx_hbm, out_hbm, idx_v,
             rows0, rows1, rows2, rows3, sem0, sem1, sem2, sem3):
        wid = lax.axis_index("s") * info.num_cores + lax.axis_index("c")
        pltpu.sync_copy(idx_hbm.at[wid], idx_v)
        bufs = (rows0, rows1, rows2, rows3)
        sems = (sem0, sem1, sem2, sem3)
        ND = 4

        def start(ci, b):
            @pl.when(ci < chunks)
            def _():
                pltpu.make_async_copy(
                    table_hbm.at[idx_v.at[ci]], bufs[b], sems[b]).start()

        def drain_store(ci, b):
            @pl.when(ci < chunks)
            def _():
                pltpu.make_async_copy(
                    table_hbm.at[idx_v.at[ci]], bufs[b], sems[b]).wait()
                pltpu.sync_copy(
                    bufs[b],
                    out_hbm.at[pl.ds((wid * chunks + ci) * 128, 128)])

        for b in range(ND):
            start(b, b)

        def quad(cq, _):
            base = cq * ND
            for b in range(ND):
                drain_store(base + b, b)
                start(base + b + ND, b)
            return 0

        lax.fori_loop(0, (chunks + ND - 1) // ND, quad, 0)

    return kern(table, idx.reshape(nw, chunks, 128))


def _k3_body(cand_ref, sub1t_ref, idx2t_ref, gbt_ref, cur_s):
    i = pl.program_id(0)
    NCH = 4
    rows = NG // NCH
    for c in range(NCH):
        blk = jnp.transpose(
            cand_ref[:, c * rows * 16:(c + 1) * rows * 16])
        cur_s[c * rows:(c + 1) * rows, :] = jnp.max(
            blk.reshape(rows, 16, BB), axis=1)
    idx2t_ref[...] = jnp.zeros((S2, BB), jnp.int32)
    li = lax.broadcasted_iota(jnp.int32, (NG, BB), 0)
    lo = lax.broadcasted_iota(jnp.int32, (S2, BB), 0)

    def body(t, _):
        cur = cur_s[...]
        m = jnp.max(cur, axis=0, keepdims=True)
        im = jnp.min(jnp.where(cur >= m, li, BIGI), axis=0, keepdims=True)
        idx2t_ref[...] += jnp.where(lo == t, im, 0)
        cur_s[...] = jnp.where(li == im, NEG, cur)
        return 0

    lax.fori_loop(0, S2, body, 0)

    sub2 = idx2t_ref[...]
    g_hi = sub2 // 8
    g_lo = sub2 % 8
    sub1 = sub1t_ref[...]
    UCH = 12
    parts = []
    for c in range(S2 // UCH):
        gh = g_hi[c * UCH:(c + 1) * UCH, :]
        oh = gh[:, None, :] == lax.broadcasted_iota(
            jnp.int32, (1, S1, 1), 1)
        parts.append(jnp.sum(jnp.where(oh, sub1[None, :, :], 0), axis=1))
    s1sel = jnp.concatenate(parts, axis=0)
    gbt_ref[...] = s1sel * 128 + g_lo * 16
    br = i * BB + lax.broadcasted_iota(jnp.int32, (S2, BB), 1)
    idx2t_ref[...] = br * S1 + g_hi


def _k3(cand_nat, sub1t):
    return pl.pallas_call(
        _k3_body,
        grid=(NBB,),
        in_specs=[
            pl.BlockSpec((BB, S1 * 128), lambda i: (i, 0)),
            pl.BlockSpec((S1, BB), lambda i: (0, i)),
        ],
        out_specs=[
            pl.BlockSpec((S2, BB), lambda i: (0, i)),
            pl.BlockSpec((S2, BB), lambda i: (0, i)),
        ],
        out_shape=[
            jax.ShapeDtypeStruct((S2, B), jnp.int32),
            jax.ShapeDtypeStruct((S2, B), jnp.int32),
        ],
        scratch_shapes=[pltpu.VMEM((NG, BB), jnp.float32)],
    )(cand_nat, sub1t)


def _k4_body(cand2t_ref, gbt_ref, tv_ref, ti_ref, attn_ref, vk_ref,
             cur_s, gidx_s):
    gb = gbt_ref[...]
    g_lo = (gb // 16) % 8
    UCH = 12
    for c in range(S2 // UCH):
        cw = jnp.transpose(
            cand2t_ref[:, c * UCH * 128:(c + 1) * UCH * 128]).reshape(
                UCH, 8, 16, BB)
        sel = (g_lo[c * UCH:(c + 1) * UCH, None, :] ==
               lax.broadcasted_iota(jnp.int32, (1, 8, 1), 1))
        cc = jnp.max(jnp.where(sel[:, :, None, :], cw, NEG), axis=1)
        cur_s[c * UCH * 16:(c + 1) * UCH * 16, :] = cc.reshape(UCH * 16, BB)
    t16 = lax.broadcasted_iota(jnp.int32, (S2, 16, BB), 1)
    gidx_s[...] = (gb[:, None, :] + t16).reshape(CAND, BB)
    tv_ref[...] = jnp.zeros((K, BB), jnp.float32)
    ti_ref[...] = jnp.zeros((K, BB), jnp.int32)
    lo = lax.broadcasted_iota(jnp.int32, (K, BB), 0)

    def body(t, _):
        cur = cur_s[...]
        gidx = gidx_s[...]
        m = jnp.max(cur, axis=0, keepdims=True)
        gi = jnp.min(jnp.where(cur >= m, gidx, BIGI), axis=0, keepdims=True)
        sel = lo == t
        tv_ref[...] += jnp.where(sel, m, 0.0)
        ti_ref[...] += jnp.where(sel, gi, 0)
        cur_s[...] = jnp.where(gidx == gi, NEG, cur)
        return 0

    lax.fori_loop(0, K, body, 0)

    tv = tv_ref[...]
    e = jnp.exp(tv - tv[0:1, :])
    attn_ref[...] = e / jnp.sum(e, axis=0, keepdims=True)
    vk_ref[...] = tv[K - 1:K, :]


def _k4(cand2_t, gbt):
    return pl.pallas_call(
        _k4_body,
        grid=(NBB,),
        in_specs=[
            pl.BlockSpec((BB, S2 * 128), lambda i: (i, 0)),
            pl.BlockSpec((S2, BB), lambda i: (0, i)),
        ],
        out_specs=[
            pl.BlockSpec((K, BB), lambda i: (0, i)),
            pl.BlockSpec((K, BB), lambda i: (0, i)),
            pl.BlockSpec((K, BB), lambda i: (0, i)),
            pl.BlockSpec((1, BB), lambda i: (0, i)),
        ],
        out_shape=[
            jax.ShapeDtypeStruct((K, B), jnp.float32),
            jax.ShapeDtypeStruct((K, B), jnp.int32),
            jax.ShapeDtypeStruct((K, B), jnp.float32),
            jax.ShapeDtypeStruct((1, B), jnp.float32),
        ],
        scratch_shapes=[
            pltpu.VMEM((CAND, BB), jnp.float32),
            pltpu.VMEM((CAND, BB), jnp.int32),
        ],
    )(cand2_t, gbt)


RB = 8192


def _k5_body(rows_ref, wv_ref, bv_ref, attn_ref, par_ref, out_ref):
    nb = RB // K
    r3 = rows_ref[...].reshape(nb, K, 2 * D)
    sel = par_ref[...][:, :, None] == 1
    cbr = jnp.where(sel, r3[:, :, D:], r3[:, :, :D]).reshape(RB, D)
    v = lax.dot_general(
        cbr, wv_ref[...],
        dimension_numbers=(((1,), (1,)), ((), ())),
        preferred_element_type=jnp.float32) + bv_ref[...]
    attn_t = jnp.tile(attn_ref[...], (1, nb))
    seg = lax.broadcasted_iota(jnp.int32, (nb, RB), 1) // K
    bl = lax.broadcasted_iota(jnp.int32, (nb, RB), 0)
    w = jnp.where(seg == bl, attn_t, 0.0)
    out_ref[...] = lax.dot_general(
        w, v, dimension_numbers=(((1,), (0,)), ((), ())),
        preferred_element_type=jnp.float32)


def _k5(rows, wv, bv, attn, par):
    nb = (B * K) // RB
    return pl.pallas_call(
        _k5_body,
        grid=(nb,),
        in_specs=[
            pl.BlockSpec((RB, 2 * D), lambda i: (i, 0)),
            pl.BlockSpec((D, D), lambda i: (0, 0)),
            pl.BlockSpec((1, D), lambda i: (0, 0)),
            pl.BlockSpec((RB // K, K), lambda i: (i, 0)),
            pl.BlockSpec((RB // K, K), lambda i: (i, 0)),
        ],
        out_specs=pl.BlockSpec((RB // K, D), lambda i: (i, 0)),
        out_shape=jax.ShapeDtypeStruct((B, D), jnp.float32),
    )(rows, wv, bv, attn, par)


def _k6_body(x_ref, cb_ref, wq_ref, bq_ref, wk_ref, bk_ref, vk_ref,
             out_ref, q_s):
    j = pl.program_id(0)

    @pl.when(j == 0)
    def _():
        q_s[...] = lax.dot_general(
            x_ref[...], wq_ref[...],
            dimension_numbers=(((1,), (1,)), ((), ())),
            preferred_element_type=jnp.float32) + bq_ref[...]

    k_t = lax.dot_general(
        cb_ref[...], wk_ref[...],
        dimension_numbers=(((1,), (1,)), ((), ())),
        preferred_element_type=jnp.float32) + bk_ref[...]
    dots = lax.dot_general(
        q_s[...], k_t,
        dimension_numbers=(((1,), (1,)), ((), ())),
        preferred_element_type=jnp.float32)
    out_ref[...] = jnp.where(dots < vk_ref[...], NEG, dots)


def _k6(x, cb, wq, bq, wk, bk, vk):
    return pl.pallas_call(
        _k6_body,
        grid=(NT,),
        in_specs=[
            pl.BlockSpec((B, D), lambda j: (0, 0)),
            pl.BlockSpec((T, D), lambda j: (j, 0)),
            pl.BlockSpec((D, D), lambda j: (0, 0)),
            pl.BlockSpec((1, D), lambda j: (0, 0)),
            pl.BlockSpec((D, D), lambda j: (0, 0)),
            pl.BlockSpec((1, D), lambda j: (0, 0)),
            pl.BlockSpec((B, 1), lambda j: (0, 0)),
        ],
        out_specs=pl.BlockSpec((B, T), lambda j: (0, j)),
        out_shape=jax.ShapeDtypeStruct((B, V), jnp.float32),
        scratch_shapes=[pltpu.VMEM((B, D), jnp.float32)],
    )(x, cb, wq, bq, wk, bk, vk)


def kernel(x, codebook, Wq, bq, Wk, bk, Wv, bv):
    dots_scratch, sub1t = _k1(x, codebook, Wq, bq.reshape(1, D),
                              Wk, bk.reshape(1, D))
    sub1 = sub1t.T
    idx1 = sub1 + jnp.arange(B, dtype=jnp.int32)[:, None] * NS1
    cand = _sc_gather(dots_scratch.reshape(B * NS1, 128),
                      idx1.reshape(B * S1), 128, B * S1)
    idx2t, gbt = _k3(cand.reshape(B, S1 * 128), sub1t)
    cand2 = _sc_gather(cand, idx2t.T.reshape(B * S2), 128, B * S2)
    tvt, tit, attnt, vkt = _k4(cand2.reshape(B, S2 * 128), gbt)
    top_value, top_ind, attn, vk = tvt.T, tit.T, attnt.T, vkt.T
    cb2 = codebook.reshape(V // 2, 2 * D)
    rows = _sc_gather(cb2, (top_ind // 2).reshape(B * K), 2 * D, B * K)
    out = _k5(rows, Wv, bv.reshape(1, D), attn, top_ind % 2)
    dots_masked = _k6(x, codebook, Wq, bq.reshape(1, D), Wk, bk.reshape(1, D),
                      vk)
    return (out, top_value, top_ind, dots_masked)


# speedup vs baseline: 1.0486x; 1.0066x over previous
import functools

import jax
import jax.numpy as jnp
from jax import lax
from jax.experimental import pallas as pl
from jax.experimental.pallas import tpu as pltpu
from jax.experimental.pallas import tpu_sc as plsc

B = 1024
D = 64
V = 100000
K = 64
T = 3072
NT = 33
VP = NT * T
NS1 = VP // 128
S1 = 72
NG = S1 * 128 // 16
S2 = 72
CAND = S2 * 16
NEG = float(-jnp.finfo(jnp.float32).max)
BIGI = 2**30

BB = 512
NBB = B // BB


def _k1_body(x_ref, cb_ref, wq_ref, bq_ref, wk_ref, bk_ref,
             dots_ref, sub1t_ref, q_s, m1t_s):
    j = pl.program_id(0)

    @pl.when(j == 0)
    def _():
        q_s[...] = lax.dot_general(
            x_ref[...], wq_ref[...],
            dimension_numbers=(((1,), (1,)), ((), ())),
            preferred_element_type=jnp.float32) + bq_ref[...]

    k_t = lax.dot_general(
        cb_ref[...], wk_ref[...],
        dimension_numbers=(((1,), (1,)), ((), ())),
        preferred_element_type=jnp.float32) + bk_ref[...]
    dots = lax.dot_general(
        q_s[...], k_t,
        dimension_numbers=(((1,), (1,)), ((), ())),
        preferred_element_type=jnp.float32)

    @pl.when(j < NT - 1)
    def _():
        dots_ref[...] = dots
        m1t_s[pl.ds(j * (T // 128), T // 128), :] = jnp.transpose(
            jnp.max(dots.reshape(B, T // 128, 128), axis=2))

    @pl.when(j == NT - 1)
    def _():
        col = j * T + lax.broadcasted_iota(jnp.int32, (B, T), 1)
        dm = jnp.where(col < V, dots, NEG)
        dots_ref[...] = dm
        m1t_s[pl.ds(j * (T // 128), T // 128), :] = jnp.transpose(
            jnp.max(dm.reshape(B, T // 128, 128), axis=2))

        sub1t_ref[...] = jnp.zeros((S1, B), jnp.int32)
        li = lax.broadcasted_iota(jnp.int32, (NS1, B), 0)
        lo = lax.broadcasted_iota(jnp.int32, (S1, B), 0)

        def body(t, _):
            cur = m1t_s[...]
            m = jnp.max(cur, axis=0, keepdims=True)
            im = jnp.min(jnp.where(cur >= m, li, BIGI), axis=0, keepdims=True)
            sub1t_ref[...] += jnp.where(lo == t, im, 0)
            m1t_s[...] = jnp.where(li == im, NEG, cur)
            return 0

        lax.fori_loop(0, S1, body, 0)


def _k1(x, cb, wq, bq, wk, bk):
    return pl.pallas_call(
        _k1_body,
        grid=(NT,),
        in_specs=[
            pl.BlockSpec((B, D), lambda j: (0, 0)),
            pl.BlockSpec((T, D), lambda j: (j, 0)),
            pl.BlockSpec((D, D), lambda j: (0, 0)),
            pl.BlockSpec((1, D), lambda j: (0, 0)),
            pl.BlockSpec((D, D), lambda j: (0, 0)),
            pl.BlockSpec((1, D), lambda j: (0, 0)),
        ],
        out_specs=[
            pl.BlockSpec((B, T), lambda j: (0, j)),
            pl.BlockSpec((S1, B), lambda j: (0, 0)),
        ],
        out_shape=[
            jax.ShapeDtypeStruct((B, VP), jnp.float32),
            jax.ShapeDtypeStruct((S1, B), jnp.int32),
        ],
        scratch_shapes=[
            pltpu.VMEM((B, D), jnp.float32),
            pltpu.VMEM((NS1, B), jnp.float32),
        ],
    )(x, cb, wq, bq, wk, bk)


def _sc_gather(table, idx, row_w, n_idx):
    info = plsc.get_sparse_core_info()
    nw = info.num_cores * info.num_subcores
    per_w = n_idx // nw
    chunks = per_w // 128
    mesh = plsc.VectorSubcoreMesh(core_axis_name="c", subcore_axis_name="s")

    @functools.partial(
        pl.kernel, mesh=mesh,
        out_type=jax.ShapeDtypeStruct((n_idx, row_w), jnp.float32),
        scratch_types=[
            pltpu.VMEM((chunks, 128), jnp.int32),
            pltpu.VMEM((128, row_w), jnp.float32),
            pltpu.VMEM((128, row_w), jnp.float32),
            pltpu.VMEM((128, row_w), jnp.float32),
            pltpu.VMEM((128, row_w), jnp.float32),
            pltpu.SemaphoreType.DMA,
            pltpu.SemaphoreType.DMA,
            pltpu.SemaphoreType.DMA,
            pltpu.SemaphoreType.DMA,
        ],
    )
    def kern(table_hbm, idx_hbm, out_hbm, idx_v,
             rows0, rows1, rows2, rows3, sem0, sem1, sem2, sem3):
        wid = lax.axis_index("s") * info.num_cores + lax.axis_index("c")
        pltpu.sync_copy(idx_hbm.at[wid], idx_v)
        bufs = (rows0, rows1, rows2, rows3)
        sems = (sem0, sem1, sem2, sem3)
        ND = 4

        def start(ci, b):
            @pl.when(ci < chunks)
            def _():
                pltpu.make_async_copy(
                    table_hbm.at[idx_v.at[ci]], bufs[b], sems[b]).start()

        def drain_store(ci, b):
            @pl.when(ci < chunks)
            def _():
                pltpu.make_async_copy(
                    table_hbm.at[idx_v.at[ci]], bufs[b], sems[b]).wait()
                pltpu.sync_copy(
                    bufs[b],
                    out_hbm.at[pl.ds((wid * chunks + ci) * 128, 128)])

        for b in range(ND):
            start(b, b)

        def quad(cq, _):
            base = cq * ND
            for b in range(ND):
                drain_store(base + b, b)
                start(base + b + ND, b)
            return 0

        lax.fori_loop(0, (chunks + ND - 1) // ND, quad, 0)

    return kern(table, idx.reshape(nw, chunks, 128))


def _k3_body(cand_ref, sub1t_ref, idx2t_ref, gbt_ref, cur_s):
    i = pl.program_id(0)
    NCH = 4
    rows = NG // NCH
    for c in range(NCH):
        blk = jnp.transpose(
            cand_ref[:, c * rows * 16:(c + 1) * rows * 16])
        cur_s[c * rows:(c + 1) * rows, :] = jnp.max(
            blk.reshape(rows, 16, BB), axis=1)
    idx2t_ref[...] = jnp.zeros((S2, BB), jnp.int32)
    li = lax.broadcasted_iota(jnp.int32, (NG, BB), 0)
    lo = lax.broadcasted_iota(jnp.int32, (S2, BB), 0)

    def body(t, _):
        cur = cur_s[...]
        m = jnp.max(cur, axis=0, keepdims=True)
        im = jnp.min(jnp.where(cur >= m, li, BIGI), axis=0, keepdims=True)
        idx2t_ref[...] += jnp.where(lo == t, im, 0)
        cur_s[...] = jnp.where(li == im, NEG, cur)
        return 0

    lax.fori_loop(0, S2, body, 0)

    sub2 = idx2t_ref[...]
    g_hi = sub2 // 8
    g_lo = sub2 % 8
    sub1 = sub1t_ref[...]
    UCH = 12
    parts = []
    for c in range(S2 // UCH):
        gh = g_hi[c * UCH:(c + 1) * UCH, :]
        oh = gh[:, None, :] == lax.broadcasted_iota(
            jnp.int32, (1, S1, 1), 1)
        parts.append(jnp.sum(jnp.where(oh, sub1[None, :, :], 0), axis=1))
    s1sel = jnp.concatenate(parts, axis=0)
    gbt_ref[...] = s1sel * 128 + g_lo * 16
    br = i * BB + lax.broadcasted_iota(jnp.int32, (S2, BB), 1)
    idx2t_ref[...] = br * S1 + g_hi


def _k3(cand_nat, sub1t):
    return pl.pallas_call(
        _k3_body,
        grid=(NBB,),
        in_specs=[
            pl.BlockSpec((BB, S1 * 128), lambda i: (i, 0)),
            pl.BlockSpec((S1, BB), lambda i: (0, i)),
        ],
        out_specs=[
            pl.BlockSpec((S2, BB), lambda i: (0, i)),
            pl.BlockSpec((S2, BB), lambda i: (0, i)),
        ],
        out_shape=[
            jax.ShapeDtypeStruct((S2, B), jnp.int32),
            jax.ShapeDtypeStruct((S2, B), jnp.int32),
        ],
        scratch_shapes=[pltpu.VMEM((NG, BB), jnp.float32)],
    )(cand_nat, sub1t)


def _k4_body(cand2t_ref, gbt_ref, tv_ref, ti_ref, attn_ref, vk_ref,
             cur_s, gidx_s):
    gb = gbt_ref[...]
    g_lo = (gb // 16) % 8
    UCH = 12
    for c in range(S2 // UCH):
        cw = jnp.transpose(
            cand2t_ref[:, c * UCH * 128:(c + 1) * UCH * 128]).reshape(
                UCH, 8, 16, BB)
        sel = (g_lo[c * UCH:(c + 1) * UCH, None, :] ==
               lax.broadcasted_iota(jnp.int32, (1, 8, 1), 1))
        cc = jnp.max(jnp.where(sel[:, :, None, :], cw, NEG), axis=1)
        cur_s[c * UCH * 16:(c + 1) * UCH * 16, :] = cc.reshape(UCH * 16, BB)
    t16 = lax.broadcasted_iota(jnp.int32, (S2, 16, BB), 1)
    gidx_s[...] = (gb[:, None, :] + t16).reshape(CAND, BB)
    tv_ref[...] = jnp.zeros((K, BB), jnp.float32)
    ti_ref[...] = jnp.zeros((K, BB), jnp.int32)
    lo = lax.broadcasted_iota(jnp.int32, (K, BB), 0)

    def body(t, _):
        cur = cur_s[...]
        gidx = gidx_s[...]
        m = jnp.max(cur, axis=0, keepdims=True)
        gi = jnp.min(jnp.where(cur >= m, gidx, BIGI), axis=0, keepdims=True)
        sel = lo == t
        tv_ref[...] += jnp.where(sel, m, 0.0)
        ti_ref[...] += jnp.where(sel, gi, 0)
        cur_s[...] = jnp.where(gidx == gi, NEG, cur)
        return 0

    lax.fori_loop(0, K, body, 0)

    tv = tv_ref[...]
    e = jnp.exp(tv - tv[0:1, :])
    attn_ref[...] = e / jnp.sum(e, axis=0, keepdims=True)
    vk_ref[...] = tv[K - 1:K, :]


def _k4(cand2_t, gbt):
    return pl.pallas_call(
        _k4_body,
        grid=(NBB,),
        in_specs=[
            pl.BlockSpec((BB, S2 * 128), lambda i: (i, 0)),
            pl.BlockSpec((S2, BB), lambda i: (0, i)),
        ],
        out_specs=[
            pl.BlockSpec((K, BB), lambda i: (0, i)),
            pl.BlockSpec((K, BB), lambda i: (0, i)),
            pl.BlockSpec((K, BB), lambda i: (0, i)),
            pl.BlockSpec((1, BB), lambda i: (0, i)),
        ],
        out_shape=[
            jax.ShapeDtypeStruct((K, B), jnp.float32),
            jax.ShapeDtypeStruct((K, B), jnp.int32),
            jax.ShapeDtypeStruct((K, B), jnp.float32),
            jax.ShapeDtypeStruct((1, B), jnp.float32),
        ],
        scratch_shapes=[
            pltpu.VMEM((CAND, BB), jnp.float32),
            pltpu.VMEM((CAND, BB), jnp.int32),
        ],
    )(cand2_t, gbt)


RB = 8192


def _k5_body(rows_ref, wv_ref, bv_ref, attn_ref, par_ref, out_ref):
    nb = RB // K
    r3 = rows_ref[...].reshape(nb, K, 2 * D)
    sel = par_ref[...][:, :, None] == 1
    cbr = jnp.where(sel, r3[:, :, D:], r3[:, :, :D]).reshape(RB, D)
    v = lax.dot_general(
        cbr, wv_ref[...],
        dimension_numbers=(((1,), (1,)), ((), ())),
        preferred_element_type=jnp.float32) + bv_ref[...]
    attn_t = jnp.tile(attn_ref[...], (1, nb))
    seg = lax.broadcasted_iota(jnp.int32, (nb, RB), 1) // K
    bl = lax.broadcasted_iota(jnp.int32, (nb, RB), 0)
    w = jnp.where(seg == bl, attn_t, 0.0)
    out_ref[...] = lax.dot_general(
        w, v, dimension_numbers=(((1,), (0,)), ((), ())),
        preferred_element_type=jnp.float32)


def _k5(rows, wv, bv, attn, par):
    nb = (B * K) // RB
    return pl.pallas_call(
        _k5_body,
        grid=(nb,),
        in_specs=[
            pl.BlockSpec((RB, 2 * D), lambda i: (i, 0)),
            pl.BlockSpec((D, D), lambda i: (0, 0)),
            pl.BlockSpec((1, D), lambda i: (0, 0)),
            pl.BlockSpec((RB // K, K), lambda i: (i, 0)),
            pl.BlockSpec((RB // K, K), lambda i: (i, 0)),
        ],
        out_specs=pl.BlockSpec((RB // K, D), lambda i: (i, 0)),
        out_shape=jax.ShapeDtypeStruct((B, D), jnp.float32),
    )(rows, wv, bv, attn, par)


def _k6_body(x_ref, cb_ref, wq_ref, bq_ref, wk_ref, bk_ref, vk_ref,
             out_ref, q_s):
    j = pl.program_id(0)

    @pl.when(j == 0)
    def _():
        q_s[...] = lax.dot_general(
            x_ref[...], wq_ref[...],
            dimension_numbers=(((1,), (1,)), ((), ())),
            preferred_element_type=jnp.float32) + bq_ref[...]

    k_t = lax.dot_general(
        cb_ref[...], wk_ref[...],
        dimension_numbers=(((1,), (1,)), ((), ())),
        preferred_element_type=jnp.float32) + bk_ref[...]
    dots = lax.dot_general(
        q_s[...], k_t,
        dimension_numbers=(((1,), (1,)), ((), ())),
        preferred_element_type=jnp.float32)
    out_ref[...] = jnp.where(dots < vk_ref[...], NEG, dots)


def _k6(x, cb, wq, bq, wk, bk, vk):
    return pl.pallas_call(
        _k6_body,
        grid=(NT,),
        in_specs=[
            pl.BlockSpec((B, D), lambda j: (0, 0)),
            pl.BlockSpec((T, D), lambda j: (j, 0)),
            pl.BlockSpec((D, D), lambda j: (0, 0)),
            pl.BlockSpec((1, D), lambda j: (0, 0)),
            pl.BlockSpec((D, D), lambda j: (0, 0)),
            pl.BlockSpec((1, D), lambda j: (0, 0)),
            pl.BlockSpec((B, 1), lambda j: (0, 0)),
        ],
        out_specs=pl.BlockSpec((B, T), lambda j: (0, j)),
        out_shape=jax.ShapeDtypeStruct((B, V), jnp.float32),
        scratch_shapes=[pltpu.VMEM((B, D), jnp.float32)],
    )(x, cb, wq, bq, wk, bk, vk)


def kernel(x, codebook, Wq, bq, Wk, bk, Wv, bv):
    dots_scratch, sub1t = _k1(x, codebook, Wq, bq.reshape(1, D),
                              Wk, bk.reshape(1, D))
    sub1 = sub1t.T
    idx1 = sub1 + jnp.arange(B, dtype=jnp.int32)[:, None] * NS1
    cand = _sc_gather(dots_scratch.reshape(B * NS1, 128),
                      idx1.reshape(B * S1), 128, B * S1)
    idx2t, gbt = _k3(cand.reshape(B, S1 * 128), sub1t)
    cand2 = _sc_gather(cand, idx2t.T.reshape(B * S2), 128, B * S2)
    tvt, tit, attnt, vkt = _k4(cand2.reshape(B, S2 * 128), gbt)
    top_value, top_ind, attn, vk = tvt.T, tit.T, attnt.T, vkt.T
    cb2 = codebook.reshape(V // 2, 2 * D)
    rows = _sc_gather(cb2, (top_ind // 2).reshape(B * K), 2 * D, B * K)
    out = _k5(rows, Wv, bv.reshape(1, D), attn, top_ind % 2)
    dots_masked = _k6(x, codebook, Wq, bq.reshape(1, D), Wk, bk.reshape(1, D),
                      vk)
    return (out, top_value, top_ind, dots_masked)
